# baseline (speedup 1.0000x reference)
import functools

import jax
import jax.numpy as jnp
from jax.experimental import pallas as pl
from jax.experimental.pallas import tpu as pltpu
from jax.experimental.pallas import tpu_sc as plsc

_WINDOW = 128


def kernel(x, weight):
    rows, cols = x.shape
    num_idx = rows * cols
    vocab, dim = weight.shape
    nwin = num_idx // _WINDOW
    xf = x.reshape(nwin, _WINDOW).astype(jnp.int32)
    qidx = xf >> 1
    w2 = weight.reshape(vocab // 2, 2 * dim)

    mesh = plsc.VectorSubcoreMesh(core_axis_name="c", subcore_axis_name="s")

    @functools.partial(
        pl.kernel,
        out_type=jax.ShapeDtypeStruct((num_idx, 2 * dim), weight.dtype),
        mesh=mesh,
    )
    def gather_kernel(w_hbm, q_hbm, o_hbm):
        def body(q_vmem, o_vmem):
            pltpu.sync_copy(w_hbm.at[q_vmem.at[0]], o_vmem)

        pltpu.emit_pipeline(
            body,
            grid=(nwin,),
            in_specs=[pl.BlockSpec((1, _WINDOW), lambda i: (i, 0))],
            out_specs=[pl.BlockSpec((_WINDOW, 2 * dim), lambda i: (i, 0))],
            core_axis_name=("c", "s"),
            dimension_semantics=(pltpu.PARALLEL,),
        )(q_hbm, o_hbm)

    pairs = gather_kernel(w2, qidx)
    h = (x.reshape(num_idx, 1) & 1) == 0
    out = jnp.where(h, pairs[:, :dim], pairs[:, dim:])
    return out.reshape(rows, cols, dim)


# speedup vs baseline: 1.4504x; 1.4504x over previous
import functools

import jax
import jax.numpy as jnp
from jax.experimental import pallas as pl
from jax.experimental.pallas import tpu as pltpu
from jax.experimental.pallas import tpu_sc as plsc

_WINDOW = 128


def kernel(x, weight):
    rows, cols = x.shape
    num_idx = rows * cols
    vocab, dim = weight.shape
    nwin = num_idx // _WINDOW
    idx = x.T.reshape(nwin, _WINDOW).astype(jnp.int32)

    mesh = plsc.VectorSubcoreMesh(core_axis_name="c", subcore_axis_name="s")

    @functools.partial(
        pl.kernel,
        out_type=jax.ShapeDtypeStruct((num_idx, dim), weight.dtype),
        mesh=mesh,
        compiler_params=pltpu.CompilerParams(use_tc_tiling_on_sc=False),
    )
    def gather_kernel(w_hbm, i_hbm, o_hbm):
        def body(i_vmem, o_vmem):
            pltpu.sync_copy(w_hbm.at[i_vmem.at[0]], o_vmem)

        pltpu.emit_pipeline(
            body,
            grid=(nwin,),
            in_specs=[pl.BlockSpec((1, _WINDOW), lambda i: (i, 0))],
            out_specs=[pl.BlockSpec((_WINDOW, dim), lambda i: (i, 0))],
            core_axis_name=("c", "s"),
            dimension_semantics=(pltpu.PARALLEL,),
        )(i_hbm, o_hbm)

    out = gather_kernel(weight, idx)
    return out.reshape(cols, rows, dim).transpose(1, 0, 2)


# speedup vs baseline: 1.4974x; 1.0324x over previous
import functools

import jax
import jax.numpy as jnp
from jax.experimental import pallas as pl
from jax.experimental.pallas import tpu as pltpu
from jax.experimental.pallas import tpu_sc as plsc

_WINDOW = 128
_GROUP = 2048


def kernel(x, weight):
    rows, cols = x.shape
    num_idx = rows * cols
    vocab, dim = weight.shape
    nwin = num_idx // _WINDOW
    half = _GROUP // 2
    idx = (
        x.T.reshape(cols, rows // _GROUP, 2, half)
        .swapaxes(2, 3)
        .reshape(nwin, _WINDOW)
        .astype(jnp.int32)
    )

    mesh = plsc.VectorSubcoreMesh(core_axis_name="c", subcore_axis_name="s")

    @functools.partial(
        pl.kernel,
        out_type=jax.ShapeDtypeStruct((num_idx, dim), weight.dtype),
        mesh=mesh,
        compiler_params=pltpu.CompilerParams(use_tc_tiling_on_sc=False),
    )
    def gather_kernel(w_hbm, i_hbm, o_hbm):
        def body(i_vmem, o_vmem):
            pltpu.sync_copy(w_hbm.at[i_vmem.at[0]], o_vmem)

        pltpu.emit_pipeline(
            body,
            grid=(nwin,),
            in_specs=[pl.BlockSpec((1, _WINDOW), lambda i: (i, 0))],
            out_specs=[pl.BlockSpec((_WINDOW, dim), lambda i: (i, 0))],
            core_axis_name=("c", "s"),
            dimension_semantics=(pltpu.PARALLEL,),
        )(i_hbm, o_hbm)

    def transpose_body(in_ref, out_ref):
        t = in_ref[0].T
        out_ref[0] = jnp.concatenate([t[:dim], t[dim:]], axis=1)

    transpose_kernel = pl.pallas_call(
        transpose_body,
        grid=(cols, rows // _GROUP),
        in_specs=[pl.BlockSpec((1, half, 2 * dim), lambda j, b: (j, b, 0))],
        out_specs=pl.BlockSpec((1, dim, _GROUP), lambda j, b: (j, 0, b)),
        out_shape=jax.ShapeDtypeStruct((cols, dim, rows), weight.dtype),
    )

    out = gather_kernel(weight, idx)
    out3 = transpose_kernel(out.reshape(cols, rows * dim // 128, 2 * dim))
    return out3.transpose(2, 0, 1)


# speedup vs baseline: 1.8694x; 1.2484x over previous
import functools

import jax
import jax.numpy as jnp
from jax import lax
from jax.experimental import pallas as pl
from jax.experimental.pallas import tpu as pltpu
from jax.experimental.pallas import tpu_sc as plsc

_WINDOW = 128
_CHUNK = _WINDOW // 2
_GROUP = 2048
_LANES = 16


def kernel(x, weight):
    rows, cols = x.shape
    num_idx = rows * cols
    vocab, dim = weight.shape
    nwin = num_idx // _WINDOW
    half = _GROUP // 2
    ngrp_w = (vocab + _GROUP - 1) // _GROUP

    xt = x.T.astype(jnp.int32)
    wt = weight.T

    xv = (xt & ~jnp.int32(_GROUP - 1)) | ((xt & (half - 1)) << 1) | (
        (xt >> 10) & 1
    )
    idx3 = xv.reshape(cols, rows // _CHUNK, _CHUNK)

    def wprep_body(in_ref, out_ref):
        t = in_ref[...].T
        out_ref[...] = jnp.concatenate([t[:half], t[half:]], axis=1)

    wprep = pl.pallas_call(
        wprep_body,
        grid=(ngrp_w,),
        in_specs=[pl.BlockSpec((dim, _GROUP), lambda b: (0, b))],
        out_specs=pl.BlockSpec((half, 2 * dim), lambda b: (b, 0)),
        out_shape=jax.ShapeDtypeStruct((ngrp_w * half, 2 * dim), weight.dtype),
    )
    w_sc = wprep(wt).reshape(ngrp_w * _GROUP, dim)

    mesh = plsc.VectorSubcoreMesh(core_axis_name="c", subcore_axis_name="s")

    @functools.partial(
        pl.kernel,
        out_type=jax.ShapeDtypeStruct((num_idx, dim), weight.dtype),
        mesh=mesh,
        scratch_types=[pltpu.VMEM((_WINDOW,), jnp.int32)],
        compiler_params=pltpu.CompilerParams(
            use_tc_tiling_on_sc=False, needs_layout_passes=False
        ),
    )
    def gather_kernel(w_hbm, ia_hbm, ib_hbm, o_hbm, perm_ref):
        def body(ia_vmem, ib_vmem, o_vmem):
            @pl.loop(0, _CHUNK, step=_LANES)
            def _(c):
                pos = (lax.iota(jnp.int32, _LANES) + c) * 2
                plsc.store_scatter(
                    perm_ref, [pos], ia_vmem[0, 0, pl.ds(c, _LANES)]
                )
                plsc.store_scatter(
                    perm_ref, [pos + 1], ib_vmem[0, 0, pl.ds(c, _LANES)]
                )

            pltpu.sync_copy(w_hbm.at[perm_ref], o_vmem)

        pltpu.emit_pipeline(
            body,
            grid=(nwin,),
            in_specs=[
                pl.BlockSpec(
                    (1, 1, _CHUNK),
                    lambda i: (i // 128, 32 * ((i % 128) // 16) + i % 16, 0),
                ),
                pl.BlockSpec(
                    (1, 1, _CHUNK),
                    lambda i: (i // 128, 32 * ((i % 128) // 16) + 16 + i % 16, 0),
                ),
            ],
            out_specs=[pl.BlockSpec((_WINDOW, dim), lambda i: (i, 0))],
            core_axis_name=("c", "s"),
            dimension_semantics=(pltpu.PARALLEL,),
        )(ia_hbm, ib_hbm, o_hbm)

    def epilogue_body(in_ref, out_ref):
        t = in_ref[0].T
        out_ref[0] = jnp.concatenate([t[:dim], t[dim:]], axis=1)

    epilogue = pl.pallas_call(
        epilogue_body,
        grid=(cols, rows // _GROUP),
        in_specs=[pl.BlockSpec((1, half, 2 * dim), lambda j, b: (j, b, 0))],
        out_specs=pl.BlockSpec((1, dim, _GROUP), lambda j, b: (j, 0, b)),
        out_shape=jax.ShapeDtypeStruct((cols, dim, rows), weight.dtype),
    )

    out = gather_kernel(w_sc, idx3, idx3)
    out3 = epilogue(out.reshape(cols, rows * dim // 128, 2 * dim))
    return out3.transpose(2, 0, 1)


# speedup vs baseline: 1.8718x; 1.0013x over previous
import functools

import jax
import jax.numpy as jnp
from jax import lax
from jax.experimental import pallas as pl
from jax.experimental.pallas import tpu as pltpu
from jax.experimental.pallas import tpu_sc as plsc

_WINDOW = 128
_CHUNK = _WINDOW // 2
_GROUP = 2048
_LANES = 16


def kernel(x, weight):
    rows, cols = x.shape
    num_idx = rows * cols
    vocab, dim = weight.shape
    nwin = num_idx // _WINDOW
    half = _GROUP // 2
    ngrp_w = (vocab + _GROUP - 1) // _GROUP

    xt = x.T.astype(jnp.int32)
    wt = weight.T

    xv = (xt & ~jnp.int32(_GROUP - 1)) | ((xt & (half - 1)) << 1) | (
        (xt >> 10) & 1
    )
    idx3 = xv.reshape(cols, rows // _CHUNK, _CHUNK)

    def wprep_body(in_ref, out_ref):
        t = in_ref[...].T
        out_ref[...] = jnp.concatenate([t[:half], t[half:]], axis=1)

    wprep = pl.pallas_call(
        wprep_body,
        grid=(ngrp_w,),
        in_specs=[pl.BlockSpec((dim, _GROUP), lambda b: (0, b))],
        out_specs=pl.BlockSpec((half, 2 * dim), lambda b: (b, 0)),
        out_shape=jax.ShapeDtypeStruct((ngrp_w * half, 2 * dim), weight.dtype),
        compiler_params=pltpu.CompilerParams(
            dimension_semantics=("parallel",)
        ),
    )
    w_sc = wprep(wt).reshape(ngrp_w * _GROUP, dim)

    mesh = plsc.VectorSubcoreMesh(core_axis_name="c", subcore_axis_name="s")

    @functools.partial(
        pl.kernel,
        out_type=jax.ShapeDtypeStruct((num_idx, dim), weight.dtype),
        mesh=mesh,
        scratch_types=[pltpu.VMEM((_WINDOW,), jnp.int32)],
        compiler_params=pltpu.CompilerParams(
            use_tc_tiling_on_sc=False, needs_layout_passes=False
        ),
    )
    def gather_kernel(w_hbm, ia_hbm, ib_hbm, o_hbm, perm_ref):
        def body(ia_vmem, ib_vmem, o_vmem):
            @pl.loop(0, _CHUNK, step=_LANES)
            def _(c):
                pos = (lax.iota(jnp.int32, _LANES) + c) * 2
                plsc.store_scatter(
                    perm_ref, [pos], ia_vmem[0, 0, pl.ds(c, _LANES)]
                )
                plsc.store_scatter(
                    perm_ref, [pos + 1], ib_vmem[0, 0, pl.ds(c, _LANES)]
                )

            pltpu.sync_copy(w_hbm.at[perm_ref], o_vmem)

        pltpu.emit_pipeline(
            body,
            grid=(nwin,),
            in_specs=[
                pl.BlockSpec(
                    (1, 1, _CHUNK),
                    lambda i: (i // 128, 32 * ((i % 128) // 16) + i % 16, 0),
                ),
                pl.BlockSpec(
                    (1, 1, _CHUNK),
                    lambda i: (i // 128, 32 * ((i % 128) // 16) + 16 + i % 16, 0),
                ),
            ],
            out_specs=[pl.BlockSpec((_WINDOW, dim), lambda i: (i, 0))],
            core_axis_name=("c", "s"),
            dimension_semantics=(pltpu.PARALLEL,),
        )(ia_hbm, ib_hbm, o_hbm)

    def epilogue_body(in_ref, out_ref):
        t = in_ref[0].T
        out_ref[0] = jnp.concatenate([t[:dim], t[dim:]], axis=1)

    epilogue = pl.pallas_call(
        epilogue_body,
        grid=(cols, rows // _GROUP),
        in_specs=[pl.BlockSpec((1, half, 2 * dim), lambda j, b: (j, b, 0))],
        out_specs=pl.BlockSpec((1, dim, _GROUP), lambda j, b: (j, 0, b)),
        out_shape=jax.ShapeDtypeStruct((cols, dim, rows), weight.dtype),
        compiler_params=pltpu.CompilerParams(
            dimension_semantics=("parallel", "parallel")
        ),
    )

    out = gather_kernel(w_sc, idx3, idx3)
    out3 = epilogue(out.reshape(cols, rows * dim // 128, 2 * dim))
    return out3.transpose(2, 0, 1)


# speedup vs baseline: 2.3963x; 1.2803x over previous
import functools

import jax
import jax.numpy as jnp
from jax import lax
from jax.experimental import pallas as pl
from jax.experimental.pallas import tpu as pltpu
from jax.experimental.pallas import tpu_sc as plsc

_WINDOW = 128
_CHUNK = _WINDOW // 2
_GROUP = 2048
_LANES = 16


def kernel(x, weight):
    rows, cols = x.shape
    num_idx = rows * cols
    vocab, dim = weight.shape
    nwin = num_idx // _WINDOW
    half = _GROUP // 2
    ngrp_w = (vocab + _GROUP - 1) // _GROUP

    xt = x.T.astype(jnp.int32)
    wt = weight.T

    xv = (xt & ~jnp.int32(_GROUP - 1)) | ((xt & (half - 1)) << 1) | (
        (xt >> 10) & 1
    )
    idx3 = xv.reshape(cols, rows // _CHUNK, _CHUNK)

    nblk_w = (ngrp_w + 1) // 2

    def wprep_body(in_ref, out_ref):
        t = in_ref[...].T
        out_ref[...] = jnp.concatenate(
            [
                jnp.concatenate([t[:half], t[half : _GROUP]], axis=1),
                jnp.concatenate([t[_GROUP : _GROUP + half], t[_GROUP + half :]], axis=1),
            ],
            axis=0,
        )

    wprep = pl.pallas_call(
        wprep_body,
        grid=(nblk_w,),
        in_specs=[pl.BlockSpec((dim, 2 * _GROUP), lambda b: (0, b))],
        out_specs=pl.BlockSpec((_GROUP, 2 * dim), lambda b: (b, 0)),
        out_shape=jax.ShapeDtypeStruct((nblk_w * _GROUP, 2 * dim), weight.dtype),
        compiler_params=pltpu.CompilerParams(
            dimension_semantics=("parallel",)
        ),
    )
    w_sc = wprep(wt).reshape(nblk_w * 2 * _GROUP, dim)

    mesh = plsc.VectorSubcoreMesh(core_axis_name="c", subcore_axis_name="s")

    @functools.partial(
        pl.kernel,
        out_type=jax.ShapeDtypeStruct((num_idx, dim), weight.dtype),
        mesh=mesh,
        scratch_types=[pltpu.VMEM((_WINDOW,), jnp.int32)],
        compiler_params=pltpu.CompilerParams(
            use_tc_tiling_on_sc=False, needs_layout_passes=False
        ),
    )
    def gather_kernel(w_hbm, ia_hbm, ib_hbm, o_hbm, perm_ref):
        def body(ia_vmem, ib_vmem, o_vmem):
            @pl.loop(0, _CHUNK, step=_LANES)
            def _(c):
                pos = (lax.iota(jnp.int32, _LANES) + c) * 2
                plsc.store_scatter(
                    perm_ref, [pos], ia_vmem[0, 0, pl.ds(c, _LANES)]
                )
                plsc.store_scatter(
                    perm_ref, [pos + 1], ib_vmem[0, 0, pl.ds(c, _LANES)]
                )

            pltpu.sync_copy(w_hbm.at[perm_ref], o_vmem)

        pltpu.emit_pipeline(
            body,
            grid=(nwin,),
            in_specs=[
                pl.BlockSpec(
                    (1, 1, _CHUNK),
                    lambda i: (i // 128, 32 * ((i % 128) // 16) + i % 16, 0),
                ),
                pl.BlockSpec(
                    (1, 1, _CHUNK),
                    lambda i: (i // 128, 32 * ((i % 128) // 16) + 16 + i % 16, 0),
                ),
            ],
            out_specs=[pl.BlockSpec((_WINDOW, dim), lambda i: (i, 0))],
            core_axis_name=("c", "s"),
            dimension_semantics=(pltpu.PARALLEL,),
        )(ia_hbm, ib_hbm, o_hbm)

    def epilogue_body(in_ref, out_ref):
        t = in_ref[0].T
        out_ref[0] = jnp.concatenate(
            [t[:dim, :half], t[dim:, :half], t[:dim, half:], t[dim:, half:]],
            axis=1,
        )

    epilogue = pl.pallas_call(
        epilogue_body,
        grid=(cols, rows // (2 * _GROUP)),
        in_specs=[pl.BlockSpec((1, _GROUP, 2 * dim), lambda j, b: (j, b, 0))],
        out_specs=pl.BlockSpec((1, dim, 2 * _GROUP), lambda j, b: (j, 0, b)),
        out_shape=jax.ShapeDtypeStruct((cols, dim, rows), weight.dtype),
        compiler_params=pltpu.CompilerParams(
            dimension_semantics=("parallel", "parallel")
        ),
    )

    out = gather_kernel(w_sc, idx3, idx3)
    out3 = epilogue(out.reshape(cols, rows * dim // 128, 2 * dim))
    return out3.transpose(2, 0, 1)


# speedup vs baseline: 2.8974x; 1.2091x over previous
import functools

import jax
import jax.numpy as jnp
from jax import lax
from jax.experimental import pallas as pl
from jax.experimental.pallas import tpu as pltpu
from jax.experimental.pallas import tpu_sc as plsc

_WINDOW = 128
_CHUNK = _WINDOW // 2
_GROUP = 2048
_LANES = 16


def kernel(x, weight):
    rows, cols = x.shape
    num_idx = rows * cols
    vocab, dim = weight.shape
    nwin = num_idx // _WINDOW
    half = _GROUP // 2
    ngrp_w = (vocab + _GROUP - 1) // _GROUP

    xt = x.T.astype(jnp.int32)
    wt = weight.T

    xv = (xt & ~jnp.int32(_GROUP - 1)) | ((xt & (half - 1)) << 1) | (
        (xt >> 10) & 1
    )
    idx3 = xv.reshape(cols, rows // _CHUNK, _CHUNK)

    _KW = 4
    nblk_w = (ngrp_w + _KW - 1) // _KW

    def wprep_body(in_ref, out_ref):
        t = in_ref[...].T
        out_ref[...] = jnp.concatenate(
            [
                jnp.concatenate(
                    [
                        t[k * _GROUP : k * _GROUP + half],
                        t[k * _GROUP + half : (k + 1) * _GROUP],
                    ],
                    axis=1,
                )
                for k in range(_KW)
            ],
            axis=0,
        )

    wprep = pl.pallas_call(
        wprep_body,
        grid=(nblk_w,),
        in_specs=[pl.BlockSpec((dim, _KW * _GROUP), lambda b: (0, b))],
        out_specs=pl.BlockSpec((_KW * half, 2 * dim), lambda b: (b, 0)),
        out_shape=jax.ShapeDtypeStruct(
            (nblk_w * _KW * half, 2 * dim), weight.dtype
        ),
        compiler_params=pltpu.CompilerParams(
            dimension_semantics=("parallel",)
        ),
    )
    w_sc = wprep(wt).reshape(nblk_w * _KW * _GROUP, dim)

    mesh = plsc.VectorSubcoreMesh(core_axis_name="c", subcore_axis_name="s")

    @functools.partial(
        pl.kernel,
        out_type=jax.ShapeDtypeStruct((num_idx, dim), weight.dtype),
        mesh=mesh,
        scratch_types=[pltpu.VMEM((_WINDOW,), jnp.int32)],
        compiler_params=pltpu.CompilerParams(
            use_tc_tiling_on_sc=False, needs_layout_passes=False
        ),
    )
    def gather_kernel(w_hbm, ia_hbm, ib_hbm, o_hbm, perm_ref):
        def body(ia_vmem, ib_vmem, o_vmem):
            @pl.loop(0, _CHUNK, step=_LANES)
            def _(c):
                pos = (lax.iota(jnp.int32, _LANES) + c) * 2
                plsc.store_scatter(
                    perm_ref, [pos], ia_vmem[0, 0, pl.ds(c, _LANES)]
                )
                plsc.store_scatter(
                    perm_ref, [pos + 1], ib_vmem[0, 0, pl.ds(c, _LANES)]
                )

            pltpu.sync_copy(w_hbm.at[perm_ref], o_vmem)

        pltpu.emit_pipeline(
            body,
            grid=(nwin,),
            in_specs=[
                pl.BlockSpec(
                    (1, 1, _CHUNK),
                    lambda i: (i // 128, 32 * ((i % 128) // 16) + i % 16, 0),
                ),
                pl.BlockSpec(
                    (1, 1, _CHUNK),
                    lambda i: (i // 128, 32 * ((i % 128) // 16) + 16 + i % 16, 0),
                ),
            ],
            out_specs=[pl.BlockSpec((_WINDOW, dim), lambda i: (i, 0))],
            core_axis_name=("c", "s"),
            dimension_semantics=(pltpu.PARALLEL,),
        )(ia_hbm, ib_hbm, o_hbm)

    _KE = 4

    def epilogue_body(in_ref, out_ref):
        t = in_ref[0].T
        out_ref[0] = jnp.concatenate(
            [
                t[s, k * half : (k + 1) * half]
                for k in range(_KE)
                for s in (slice(0, dim), slice(dim, 2 * dim))
            ],
            axis=1,
        )

    epilogue = pl.pallas_call(
        epilogue_body,
        grid=(cols, rows // (_KE * _GROUP)),
        in_specs=[
            pl.BlockSpec((1, _KE * half, 2 * dim), lambda j, b: (j, b, 0))
        ],
        out_specs=pl.BlockSpec((1, dim, _KE * _GROUP), lambda j, b: (j, 0, b)),
        out_shape=jax.ShapeDtypeStruct((cols, dim, rows), weight.dtype),
        compiler_params=pltpu.CompilerParams(
            dimension_semantics=("parallel", "parallel")
        ),
    )

    out = gather_kernel(w_sc, idx3, idx3)
    out3 = epilogue(out.reshape(cols, rows * dim // 128, 2 * dim))
    return out3.transpose(2, 0, 1)


# speedup vs baseline: 3.1893x; 1.1007x over previous
import functools

import jax
import jax.numpy as jnp
from jax import lax
from jax.experimental import pallas as pl
from jax.experimental.pallas import tpu as pltpu
from jax.experimental.pallas import tpu_sc as plsc

_WINDOW = 128
_CHUNK = _WINDOW // 2
_GROUP = 2048
_LANES = 16


def kernel(x, weight):
    rows, cols = x.shape
    num_idx = rows * cols
    vocab, dim = weight.shape
    nwin = num_idx // _WINDOW
    half = _GROUP // 2
    ngrp_w = (vocab + _GROUP - 1) // _GROUP

    xt = x.T.astype(jnp.int32)
    wt = weight.T

    xv = (xt & ~jnp.int32(_GROUP - 1)) | ((xt & (half - 1)) << 1) | (
        (xt >> 10) & 1
    )
    idx3 = xv.reshape(cols, rows // _CHUNK, _CHUNK)

    _KW = 8
    nblk_w = (ngrp_w + _KW - 1) // _KW

    def wprep_body(in_ref, out_ref):
        t = in_ref[...].T
        out_ref[...] = jnp.concatenate(
            [
                jnp.concatenate(
                    [
                        t[k * _GROUP : k * _GROUP + half],
                        t[k * _GROUP + half : (k + 1) * _GROUP],
                    ],
                    axis=1,
                )
                for k in range(_KW)
            ],
            axis=0,
        )

    wprep = pl.pallas_call(
        wprep_body,
        grid=(nblk_w,),
        in_specs=[pl.BlockSpec((dim, _KW * _GROUP), lambda b: (0, b))],
        out_specs=pl.BlockSpec((_KW * half, 2 * dim), lambda b: (b, 0)),
        out_shape=jax.ShapeDtypeStruct(
            (nblk_w * _KW * half, 2 * dim), weight.dtype
        ),
        compiler_params=pltpu.CompilerParams(
            dimension_semantics=("parallel",)
        ),
    )
    w_sc = wprep(wt).reshape(nblk_w * _KW * _GROUP, dim)

    mesh = plsc.VectorSubcoreMesh(core_axis_name="c", subcore_axis_name="s")

    @functools.partial(
        pl.kernel,
        out_type=jax.ShapeDtypeStruct((num_idx, dim), weight.dtype),
        mesh=mesh,
        scratch_types=[pltpu.VMEM((_WINDOW,), jnp.int32)],
        compiler_params=pltpu.CompilerParams(
            use_tc_tiling_on_sc=False, needs_layout_passes=False
        ),
    )
    def gather_kernel(w_hbm, ia_hbm, ib_hbm, o_hbm, perm_ref):
        def body(ia_vmem, ib_vmem, o_vmem):
            @pl.loop(0, _CHUNK, step=_LANES)
            def _(c):
                pos = (lax.iota(jnp.int32, _LANES) + c) * 2
                plsc.store_scatter(
                    perm_ref, [pos], ia_vmem[0, 0, pl.ds(c, _LANES)]
                )
                plsc.store_scatter(
                    perm_ref, [pos + 1], ib_vmem[0, 0, pl.ds(c, _LANES)]
                )

            pltpu.sync_copy(w_hbm.at[perm_ref], o_vmem)

        pltpu.emit_pipeline(
            body,
            grid=(nwin,),
            in_specs=[
                pl.BlockSpec(
                    (1, 1, _CHUNK),
                    lambda i: (i // 128, 32 * ((i % 128) // 16) + i % 16, 0),
                ),
                pl.BlockSpec(
                    (1, 1, _CHUNK),
                    lambda i: (i // 128, 32 * ((i % 128) // 16) + 16 + i % 16, 0),
                ),
            ],
            out_specs=[pl.BlockSpec((_WINDOW, dim), lambda i: (i, 0))],
            core_axis_name=("c", "s"),
            dimension_semantics=(pltpu.PARALLEL,),
        )(ia_hbm, ib_hbm, o_hbm)

    _KE = 8

    def epilogue_body(in_ref, out_ref):
        t = in_ref[0].T
        out_ref[0] = jnp.concatenate(
            [
                t[s, k * half : (k + 1) * half]
                for k in range(_KE)
                for s in (slice(0, dim), slice(dim, 2 * dim))
            ],
            axis=1,
        )

    epilogue = pl.pallas_call(
        epilogue_body,
        grid=(cols, rows // (_KE * _GROUP)),
        in_specs=[
            pl.BlockSpec((1, _KE * half, 2 * dim), lambda j, b: (j, b, 0))
        ],
        out_specs=pl.BlockSpec((1, dim, _KE * _GROUP), lambda j, b: (j, 0, b)),
        out_shape=jax.ShapeDtypeStruct((cols, dim, rows), weight.dtype),
        compiler_params=pltpu.CompilerParams(
            dimension_semantics=("parallel", "parallel")
        ),
    )

    out = gather_kernel(w_sc, idx3, idx3)
    out3 = epilogue(out.reshape(cols, rows * dim // 128, 2 * dim))
    return out3.transpose(2, 0, 1)
